# baseline (speedup 1.0000x reference)
import jax
import jax.numpy as jnp
from jax import lax
from jax.experimental import pallas as pl
from jax.experimental.pallas import tpu as pltpu
from jax.experimental.pallas import tpu_sc as plsc

_D = 128
_NC = 2
_NS = 16
_NW = _NC * _NS
_CH = 64
_NBUF = 10


def _emb_body(idx_hbm, w_hbm, out_hbm, idx_v, rows_v, gsem, ssem):
    nch = idx_v.shape[0]
    ng = nch // _NBUF
    wid = lax.axis_index("s") * _NC + lax.axis_index("c")
    pltpu.sync_copy(idx_hbm.at[wid], idx_v)
    base = wid * (nch * _CH)

    def gather(j, b):
        pltpu.async_copy(w_hbm.at[idx_v.at[j]], rows_v.at[b], gsem.at[b])

    def wait_gather(b):
        pltpu.make_async_copy(w_hbm.at[idx_v.at[0]], rows_v.at[b],
                              gsem.at[b]).wait()

    def store(j, b):
        pltpu.async_copy(rows_v.at[b], out_hbm.at[pl.ds(base + j * _CH, _CH)],
                         ssem.at[b])

    def wait_store(b):
        pltpu.make_async_copy(rows_v.at[b],
                              out_hbm.at[pl.ds(base, _CH)], ssem.at[b]).wait()

    gather(0, 0)

    def outer(g, carry):
        for b in range(_NBUF):
            j = g * _NBUF + b
            bp = (b - 1) % _NBUF
            bn = (b + 1) % _NBUF

            if b == _NBUF - 1:
                def do_gather(jn=j + 1, bn=bn):
                    wait_store(bn)
                    gather(jn, bn)

                pl.when(g < ng - 1)(do_gather)
            else:
                def do_wait_store(bn=bn):
                    wait_store(bn)

                pl.when(g > 0)(do_wait_store)
                gather(j + 1, bn)

            def do_store(jp=j - 1, bp=bp):
                wait_gather(bp)
                store(jp, bp)

            if b == 0:
                pl.when(g > 0)(do_store)
            else:
                do_store()

        return carry

    lax.fori_loop(0, ng, outer, 0)
    wait_gather((nch - 1) % _NBUF)
    store(nch - 1, (nch - 1) % _NBUF)
    for b in range(_NBUF):
        wait_store(b)


@jax.jit
def _emb(idx3, w):
    nch = idx3.shape[1]
    nrows = _NW * nch * _CH
    mesh = plsc.VectorSubcoreMesh(core_axis_name="c", subcore_axis_name="s")
    f = pl.kernel(
        _emb_body,
        out_type=jax.ShapeDtypeStruct((nrows, _D), jnp.float32),
        mesh=mesh,
        scratch_types=[
            pltpu.VMEM((nch, _CH), jnp.int32),
            pltpu.VMEM((_NBUF, _CH, _D), jnp.float32),
            pltpu.SemaphoreType.DMA((_NBUF,)),
            pltpu.SemaphoreType.DMA((_NBUF,)),
        ],
    )
    return f(idx3, w)


def kernel(idx, W):
    b, h = idx.shape
    n = b * h
    idx_t = jnp.transpose(idx.astype(jnp.int32))
    idx3 = idx_t.reshape(_NW, n // (_NW * _CH), _CH)
    out = _emb(idx3, W)
    return jnp.transpose(out.reshape(h, b, _D), (1, 0, 2))


# speedup vs baseline: 1.0141x; 1.0141x over previous
import jax
import jax.numpy as jnp
from jax import lax
from jax.experimental import pallas as pl
from jax.experimental.pallas import tpu as pltpu
from jax.experimental.pallas import tpu_sc as plsc

_D = 128
_NC = 2
_NS = 16
_NW = _NC * _NS
_CH = 128
_NBUF = 5


def _emb_body(idx_hbm, w_hbm, out_hbm, idx_v, rows_v, gsem, ssem):
    nch = idx_v.shape[0]
    ng = nch // _NBUF
    wid = lax.axis_index("s") * _NC + lax.axis_index("c")
    pltpu.sync_copy(idx_hbm.at[wid], idx_v)
    base = wid * (nch * _CH)

    def gather(j, b):
        pltpu.async_copy(w_hbm.at[idx_v.at[j]], rows_v.at[b], gsem.at[b])

    def wait_gather(b):
        pltpu.make_async_copy(w_hbm.at[idx_v.at[0]], rows_v.at[b],
                              gsem.at[b]).wait()

    def store(j, b):
        pltpu.async_copy(rows_v.at[b], out_hbm.at[pl.ds(base + j * _CH, _CH)],
                         ssem.at[b])

    def wait_store(b):
        pltpu.make_async_copy(rows_v.at[b],
                              out_hbm.at[pl.ds(base, _CH)], ssem.at[b]).wait()

    gather(0, 0)

    def outer(g, carry):
        for b in range(_NBUF):
            j = g * _NBUF + b
            bp = (b - 1) % _NBUF
            bn = (b + 1) % _NBUF

            if b == _NBUF - 1:
                def do_gather(jn=j + 1, bn=bn):
                    wait_store(bn)
                    gather(jn, bn)

                pl.when(g < ng - 1)(do_gather)
            else:
                def do_wait_store(bn=bn):
                    wait_store(bn)

                pl.when(g > 0)(do_wait_store)
                gather(j + 1, bn)

            def do_store(jp=j - 1, bp=bp):
                wait_gather(bp)
                store(jp, bp)

            if b == 0:
                pl.when(g > 0)(do_store)
            else:
                do_store()

        return carry

    lax.fori_loop(0, ng, outer, 0)
    wait_gather((nch - 1) % _NBUF)
    store(nch - 1, (nch - 1) % _NBUF)
    for b in range(_NBUF):
        wait_store(b)


@jax.jit
def _emb(idx3, w):
    nch = idx3.shape[1]
    nrows = _NW * nch * _CH
    mesh = plsc.VectorSubcoreMesh(core_axis_name="c", subcore_axis_name="s")
    f = pl.kernel(
        _emb_body,
        out_type=jax.ShapeDtypeStruct((nrows, _D), jnp.float32),
        mesh=mesh,
        scratch_types=[
            pltpu.VMEM((nch, _CH), jnp.int32),
            pltpu.VMEM((_NBUF, _CH, _D), jnp.float32),
            pltpu.SemaphoreType.DMA((_NBUF,)),
            pltpu.SemaphoreType.DMA((_NBUF,)),
        ],
    )
    return f(idx3, w)


def kernel(idx, W):
    b, h = idx.shape
    n = b * h
    idx_t = jnp.transpose(idx.astype(jnp.int32))
    idx3 = idx_t.reshape(_NW, n // (_NW * _CH), _CH)
    out = _emb(idx3, W)
    return jnp.transpose(out.reshape(h, b, _D), (1, 0, 2))
